# baseline (speedup 1.0000x reference)
import functools

import jax
import jax.numpy as jnp
from jax import lax
from jax.experimental import pallas as pl
from jax.experimental.pallas import tpu as pltpu
from jax.experimental.pallas import tpu_sc as plsc

_DIM = 64
_SCALE = 8.0
_CHUNK = 128
_NW = 32
_B = 4096 * 200
_BLOCKS = _B // _CHUNK
_BLK_PER_W = _BLOCKS // _NW


def _body(x_hbm, table_hbm, out_hbm, idx_v, rows_v, gsem):
    cid = lax.axis_index("c")
    sid = lax.axis_index("s")
    wid = sid * 2 + cid
    base_blk = wid * _BLK_PER_W

    pltpu.sync_copy(x_hbm.at[pl.ds(base_blk, _BLK_PER_W)], idx_v)

    def step(j, carry):
        pltpu.async_copy(table_hbm.at[idx_v.at[j]], rows_v, gsem).wait()

        def srow(i, c2):
            for cb in range(_DIM // 16):
                sl = pl.ds(cb * 16, 16)
                rows_v[i, sl] = rows_v[i, sl] * _SCALE
            return c2

        lax.fori_loop(0, _CHUNK, srow, 0)

        pltpu.sync_copy(rows_v, out_hbm.at[pl.ds((base_blk + j) * _CHUNK, _CHUNK)])
        return carry

    lax.fori_loop(0, _BLK_PER_W, step, 0)


@jax.jit
def _embed(x2d, table):
    mesh = plsc.VectorSubcoreMesh(core_axis_name="c", subcore_axis_name="s")
    kfn = pl.kernel(
        _body,
        out_type=jax.ShapeDtypeStruct((_B, _DIM), jnp.float32),
        mesh=mesh,
        scratch_types=[
            pltpu.VMEM((_BLK_PER_W, _CHUNK), jnp.int32),
            pltpu.VMEM((_CHUNK, _DIM), jnp.float32),
            pltpu.SemaphoreType.DMA,
        ],
        compiler_params=pltpu.CompilerParams(use_tc_tiling_on_sc=False),
    )
    return kfn(x2d, table)


def kernel(x, table):
    x2d = x.reshape(_BLOCKS, _CHUNK)
    out = _embed(x2d, table)
    return out.reshape(x.shape[0], x.shape[1], _DIM)


# speedup vs baseline: 1.2026x; 1.2026x over previous
import functools

import jax
import jax.numpy as jnp
from jax import lax
from jax.experimental import pallas as pl
from jax.experimental.pallas import tpu as pltpu
from jax.experimental.pallas import tpu_sc as plsc

_DIM = 64
_SCALE = 8.0
_CHUNK = 128
_NW = 32
_B = 4096 * 200
_BLOCKS = _B // _CHUNK
_BLK_PER_W = _BLOCKS // _NW


_NBUF = 4


def _body(x_hbm, table_hbm, out_hbm, idx_v, grows, srows, gsem, ssem):
    cid = lax.axis_index("c")
    sid = lax.axis_index("s")
    wid = sid * 2 + cid
    base_blk = wid * _BLK_PER_W

    pltpu.sync_copy(x_hbm.at[pl.ds(base_blk, _BLK_PER_W)], idx_v)

    def gather_start(j, b):
        pltpu.async_copy(table_hbm.at[idx_v.at[j]], grows.at[b], gsem.at[b])

    def gather_wait(j, b):
        pltpu.make_async_copy(table_hbm.at[idx_v.at[j]], grows.at[b],
                              gsem.at[b]).wait()

    def scatter_start(j, b):
        pltpu.async_copy(srows.at[b],
                         out_hbm.at[pl.ds((base_blk + j) * _CHUNK, _CHUNK)],
                         ssem.at[b])

    def scatter_wait(j, b):
        pltpu.make_async_copy(srows.at[b],
                              out_hbm.at[pl.ds((base_blk + j) * _CHUNK, _CHUNK)],
                              ssem.at[b]).wait()

    for b in range(_NBUF):
        gather_start(b, b)

    def outer(jo, carry):
        for b in range(_NBUF):
            j = jo * _NBUF + b
            gather_wait(j, b)

            @pl.when(jo > 0)
            def _():
                scatter_wait(j - _NBUF, b)

            def srow(i, c2):
                for u in range(16):
                    r = i * 4 + u // 4
                    sl = pl.ds((u % 4) * 16, 16)
                    srows[b, r, sl] = grows[b, r, sl] * _SCALE
                return c2

            lax.fori_loop(0, _CHUNK // 4, srow, 0)

            scatter_start(j, b)

            @pl.when(j + _NBUF < _BLK_PER_W)
            def _():
                gather_start(j + _NBUF, b)
        return carry

    lax.fori_loop(0, _BLK_PER_W // _NBUF, outer, 0)

    for b in range(_NBUF):
        scatter_wait(_BLK_PER_W - _NBUF + b, b)


@jax.jit
def _embed(x2d, table):
    mesh = plsc.VectorSubcoreMesh(core_axis_name="c", subcore_axis_name="s")
    kfn = pl.kernel(
        _body,
        out_type=jax.ShapeDtypeStruct((_B, _DIM), jnp.float32),
        mesh=mesh,
        scratch_types=[
            pltpu.VMEM((_BLK_PER_W, _CHUNK), jnp.int32),
            pltpu.VMEM((_NBUF, _CHUNK, _DIM), jnp.float32),
            pltpu.VMEM((_NBUF, _CHUNK, _DIM), jnp.float32),
            pltpu.SemaphoreType.DMA((_NBUF,)),
            pltpu.SemaphoreType.DMA((_NBUF,)),
        ],
        compiler_params=pltpu.CompilerParams(use_tc_tiling_on_sc=False),
    )
    return kfn(x2d, table)


def kernel(x, table):
    x2d = x.reshape(_BLOCKS, _CHUNK)
    out = _embed(x2d, table)
    return out.reshape(x.shape[0], x.shape[1], _DIM)
